# baseline (speedup 1.0000x reference)
import functools

import jax
import jax.numpy as jnp
from jax import lax
from jax.experimental import pallas as pl
from jax.experimental.pallas import tpu as pltpu
from jax.experimental.pallas import tpu_sc as plsc

N_SITES = 6400000
N_CRYSTALS = 100000
NC, NS = 2, 16
NW = NC * NS
SITES_PER_W = N_SITES // NW
PAD_SEG = 100352
SEG_PER_TILE = PAD_SEG // NS
FW = 2048
CHUNK = 4096
NCH = SITES_PER_W // CHUNK
REM = SITES_PER_W - NCH * CHUNK
UNROLL = 8

_MESH = plsc.VectorSubcoreMesh(core_axis_name="c", subcore_axis_name="s")


@functools.partial(
    pl.kernel,
    out_type=jax.ShapeDtypeStruct((NC * PAD_SEG,), jnp.float32),
    mesh=_MESH,
    scratch_types=[
        pltpu.VMEM_SHARED((PAD_SEG,), jnp.float32),
        pltpu.VMEM((PAD_SEG,), jnp.float32),
        pltpu.VMEM((CHUNK,), jnp.int32),
        pltpu.VMEM((CHUNK,), jnp.float32),
        pltpu.VMEM((CHUNK,), jnp.int32),
        pltpu.VMEM((CHUNK,), jnp.float32),
        pltpu.VMEM((FW,), jnp.int32),
        pltpu.VMEM((16,), jnp.int32),
        pltpu.VMEM((16,), jnp.int32),
        pltpu.SemaphoreType.DMA,
        pltpu.SemaphoreType.DMA,
    ],
    compiler_params=pltpu.CompilerParams(needs_layout_passes=False),
)
def _sc_partial(en_hbm, ids_hbm, zeros_hbm, out_hbm,
                acc, lacc, ids_b0, en_b0, ids_b1, en_b1,
                idxb, fb, lb, sem_i, sem_e):
    c = lax.axis_index("c")
    s = lax.axis_index("s")
    wid = c * NS + s
    base = wid * SITES_PER_W

    pltpu.sync_copy(zeros_hbm,
                    acc.at[pl.ds(s * SEG_PER_TILE, SEG_PER_TILE)])

    pltpu.sync_copy(ids_hbm.at[pl.ds(base, 16)], fb)
    pltpu.sync_copy(ids_hbm.at[pl.ds(base + SITES_PER_W - 16, 16)], lb)
    lo = fb[...][0]
    hi = lb[...][15]
    g0 = lo // FW
    nf = hi // FW - g0 + 1
    za0 = g0 * FW

    def zbody(j, _):
        lacc[pl.ds(za0 + j * 16, 16)] = jnp.zeros((16,), jnp.float32)
        return ()
    lax.fori_loop(0, nf * (FW // 16), zbody, ())

    def process(ids_b, en_b, nvec):
        def pbody(j, _):
            for k in range(UNROLL):
                sl = pl.ds((j * UNROLL + k) * 16, 16)
                plsc.addupdate_scatter(lacc, [ids_b[sl]], en_b[sl])
            return ()
        lax.fori_loop(0, nvec // UNROLL, pbody, ())

        def tbody(j, _):
            sl = pl.ds(j * 16, 16)
            plsc.addupdate_scatter(lacc, [ids_b[sl]], en_b[sl])
            return ()
        lax.fori_loop(nvec // UNROLL * UNROLL, nvec, tbody, ())

    rst = base + NCH * CHUNK
    pltpu.sync_copy(ids_hbm.at[pl.ds(rst, REM)], ids_b0.at[pl.ds(0, REM)])
    pltpu.sync_copy(en_hbm.at[pl.ds(rst, REM)], en_b0.at[pl.ds(0, REM)])
    process(ids_b0, en_b0, REM // 16)

    def start(gi, ids_b, en_b):
        st = base + gi * CHUNK
        pltpu.async_copy(ids_hbm.at[pl.ds(st, CHUNK)], ids_b, sem_i)
        pltpu.async_copy(en_hbm.at[pl.ds(st, CHUNK)], en_b, sem_e)

    def drain(ids_b, en_b):
        pltpu.make_async_copy(ids_hbm.at[pl.ds(0, CHUNK)], ids_b, sem_i).wait()
        pltpu.make_async_copy(en_hbm.at[pl.ds(0, CHUNK)], en_b, sem_e).wait()

    bufs = ((ids_b0, en_b0), (ids_b1, en_b1))
    start(0, *bufs[0])

    def ring(g2, _):
        for b in range(2):
            g = g2 * 2 + b
            drain(*bufs[b])

            @pl.when(g + 1 < NCH)
            def _():
                start(g + 1, *bufs[1 - b])

            process(*bufs[b], CHUNK // 16)
        return ()
    lax.fori_loop(0, NCH // 2, ring, ())

    plsc.subcore_barrier()

    def fbody(g, _):
        st = (g0 + g) * FW

        def ibody(j, _):
            idxb[pl.ds(j * 16, 16)] = (
                st + j * 16 + lax.iota(jnp.int32, 16))
            return ()
        lax.fori_loop(0, FW // 16, ibody, ())
        pltpu.sync_copy(lacc.at[pl.ds(st, FW)], acc.at[idxb], add=True)
        return ()
    lax.fori_loop(0, nf, fbody, ())

    plsc.subcore_barrier()

    off = s * SEG_PER_TILE
    pltpu.sync_copy(acc.at[pl.ds(off, SEG_PER_TILE)],
                    lacc.at[pl.ds(0, SEG_PER_TILE)])
    pltpu.sync_copy(lacc.at[pl.ds(0, SEG_PER_TILE)],
                    out_hbm.at[pl.ds(c * PAD_SEG + off, SEG_PER_TILE)])


SEG_PER_W = PAD_SEG // NW


@functools.partial(
    pl.kernel,
    out_type=jax.ShapeDtypeStruct((PAD_SEG,), jnp.float32),
    mesh=_MESH,
    scratch_types=[
        pltpu.VMEM((SEG_PER_W,), jnp.float32),
        pltpu.VMEM((SEG_PER_W,), jnp.float32),
    ],
)
def _sc_combine(partial_hbm, out_hbm, buf0, buf1):
    c = lax.axis_index("c")
    s = lax.axis_index("s")
    wid = c * NS + s
    off = wid * SEG_PER_W
    pltpu.sync_copy(partial_hbm.at[pl.ds(off, SEG_PER_W)], buf0)
    pltpu.sync_copy(partial_hbm.at[pl.ds(PAD_SEG + off, SEG_PER_W)], buf1)
    for j in range(SEG_PER_W // 16):
        sl = pl.ds(j * 16, 16)
        buf0[sl] = buf0[sl] + buf1[sl]
    pltpu.sync_copy(buf0, out_hbm.at[pl.ds(off, SEG_PER_W)])


def kernel(site_energy, segment_ids, num_crystals):
    en = site_energy.reshape(N_SITES)
    ids = segment_ids.reshape(N_SITES)
    zeros = jnp.zeros((SEG_PER_TILE,), jnp.float32)
    partial = _sc_partial(en, ids, zeros)
    pooled = _sc_combine(partial)
    return pooled[:N_CRYSTALS, None]


# speedup vs baseline: 1.6015x; 1.6015x over previous
import functools

import jax
import jax.numpy as jnp
from jax import lax
from jax.experimental import pallas as pl
from jax.experimental.pallas import tpu as pltpu
from jax.experimental.pallas import tpu_sc as plsc

N_SITES = 6400000
N_CRYSTALS = 100000
NC, NS = 2, 16
NW = NC * NS
SITES_PER_W = N_SITES // NW
PAD_SEG = 100352
SEG_PER_TILE = PAD_SEG // NS
FW = 2048
CHUNK = 4096
NCH = SITES_PER_W // CHUNK
REM = SITES_PER_W - NCH * CHUNK
UNROLL = 8

_MESH = plsc.VectorSubcoreMesh(core_axis_name="c", subcore_axis_name="s")


@functools.partial(
    pl.kernel,
    out_type=jax.ShapeDtypeStruct((NC * PAD_SEG,), jnp.float32),
    mesh=_MESH,
    scratch_types=[
        pltpu.VMEM_SHARED((PAD_SEG,), jnp.float32),
        pltpu.VMEM((PAD_SEG,), jnp.float32),
        pltpu.VMEM((CHUNK + 16,), jnp.int32),
        pltpu.VMEM((CHUNK,), jnp.float32),
        pltpu.VMEM((CHUNK + 16,), jnp.int32),
        pltpu.VMEM((CHUNK,), jnp.float32),
        pltpu.VMEM((FW,), jnp.int32),
        pltpu.VMEM((16,), jnp.int32),
        pltpu.VMEM((16,), jnp.int32),
        pltpu.SemaphoreType.DMA,
        pltpu.SemaphoreType.DMA,
    ],
    compiler_params=pltpu.CompilerParams(needs_layout_passes=False),
)
def _sc_partial(en_hbm, ids_hbm, zeros_hbm, out_hbm,
                acc, lacc, ids_b0, en_b0, ids_b1, en_b1,
                idxb, fb, lb, sem_i, sem_e):
    c = lax.axis_index("c")
    s = lax.axis_index("s")
    wid = c * NS + s
    base = wid * SITES_PER_W

    pltpu.sync_copy(zeros_hbm,
                    acc.at[pl.ds(s * SEG_PER_TILE, SEG_PER_TILE)])

    pltpu.sync_copy(ids_hbm.at[pl.ds(base, 16)], fb)
    pltpu.sync_copy(ids_hbm.at[pl.ds(base + SITES_PER_W - 16, 16)], lb)
    lo = fb[...][0]
    hi = lb[...][15]
    g0 = lo // FW
    nf = hi // FW - g0 + 1
    za0 = g0 * FW

    def zbody(j, _):
        lacc[pl.ds(za0 + j * 16, 16)] = jnp.zeros((16,), jnp.float32)
        return ()
    lax.fori_loop(0, nf * (FW // 16), zbody, ())

    iota = lax.iota(jnp.int32, 16)
    lane15 = iota == 15
    not15 = iota != 15

    def vreg(ids_b, en_b, j):
        idv = ids_b[pl.ds(j * 16, 16)]
        idv_up = ids_b[pl.ds(j * 16 + 1, 16)]
        env = en_b[pl.ds(j * 16, 16)]
        m_change = (idv != idv_up) & not15
        m_end = m_change | lane15
        cs = plsc.cumsum(env)
        plsc.addupdate_scatter(lacc, [idv], cs, mask=m_end)
        plsc.addupdate_scatter(lacc, [idv_up], -cs, mask=m_change)

    def process(ids_b, en_b, nvec):
        def pbody(j, _):
            for k in range(UNROLL):
                vreg(ids_b, en_b, j * UNROLL + k)
            return ()
        lax.fori_loop(0, nvec // UNROLL, pbody, ())

        def tbody(j, _):
            vreg(ids_b, en_b, j)
            return ()
        lax.fori_loop(nvec // UNROLL * UNROLL, nvec, tbody, ())

    rst = base + NCH * CHUNK
    pltpu.sync_copy(ids_hbm.at[pl.ds(rst, REM)], ids_b0.at[pl.ds(0, REM)])
    pltpu.sync_copy(en_hbm.at[pl.ds(rst, REM)], en_b0.at[pl.ds(0, REM)])
    process(ids_b0, en_b0, REM // 16)

    def start(gi, ids_b, en_b):
        st = base + gi * CHUNK
        pltpu.async_copy(ids_hbm.at[pl.ds(st, CHUNK)],
                         ids_b.at[pl.ds(0, CHUNK)], sem_i)
        pltpu.async_copy(en_hbm.at[pl.ds(st, CHUNK)], en_b, sem_e)

    def drain(ids_b, en_b):
        pltpu.make_async_copy(ids_hbm.at[pl.ds(0, CHUNK)],
                              ids_b.at[pl.ds(0, CHUNK)], sem_i).wait()
        pltpu.make_async_copy(en_hbm.at[pl.ds(0, CHUNK)], en_b, sem_e).wait()

    bufs = ((ids_b0, en_b0), (ids_b1, en_b1))
    start(0, *bufs[0])

    def ring(g2, _):
        for b in range(2):
            g = g2 * 2 + b
            drain(*bufs[b])

            @pl.when(g + 1 < NCH)
            def _():
                start(g + 1, *bufs[1 - b])

            process(*bufs[b], CHUNK // 16)
        return ()
    lax.fori_loop(0, NCH // 2, ring, ())

    plsc.subcore_barrier()

    def fbody(g, _):
        st = (g0 + g) * FW

        def ibody(j, _):
            idxb[pl.ds(j * 16, 16)] = (
                st + j * 16 + lax.iota(jnp.int32, 16))
            return ()
        lax.fori_loop(0, FW // 16, ibody, ())
        pltpu.sync_copy(lacc.at[pl.ds(st, FW)], acc.at[idxb], add=True)
        return ()
    lax.fori_loop(0, nf, fbody, ())

    plsc.subcore_barrier()

    off = s * SEG_PER_TILE
    pltpu.sync_copy(acc.at[pl.ds(off, SEG_PER_TILE)],
                    lacc.at[pl.ds(0, SEG_PER_TILE)])
    pltpu.sync_copy(lacc.at[pl.ds(0, SEG_PER_TILE)],
                    out_hbm.at[pl.ds(c * PAD_SEG + off, SEG_PER_TILE)])


SEG_PER_W = PAD_SEG // NW


@functools.partial(
    pl.kernel,
    out_type=jax.ShapeDtypeStruct((PAD_SEG,), jnp.float32),
    mesh=_MESH,
    scratch_types=[
        pltpu.VMEM((SEG_PER_W,), jnp.float32),
        pltpu.VMEM((SEG_PER_W,), jnp.float32),
    ],
)
def _sc_combine(partial_hbm, out_hbm, buf0, buf1):
    c = lax.axis_index("c")
    s = lax.axis_index("s")
    wid = c * NS + s
    off = wid * SEG_PER_W
    pltpu.sync_copy(partial_hbm.at[pl.ds(off, SEG_PER_W)], buf0)
    pltpu.sync_copy(partial_hbm.at[pl.ds(PAD_SEG + off, SEG_PER_W)], buf1)
    for j in range(SEG_PER_W // 16):
        sl = pl.ds(j * 16, 16)
        buf0[sl] = buf0[sl] + buf1[sl]
    pltpu.sync_copy(buf0, out_hbm.at[pl.ds(off, SEG_PER_W)])


def kernel(site_energy, segment_ids, num_crystals):
    en = site_energy.reshape(N_SITES)
    ids = segment_ids.reshape(N_SITES)
    zeros = jnp.zeros((SEG_PER_TILE,), jnp.float32)
    partial = _sc_partial(en, ids, zeros)
    pooled = _sc_combine(partial)
    return pooled[:N_CRYSTALS, None]


# speedup vs baseline: 3.5493x; 2.2162x over previous
import functools

import jax
import jax.numpy as jnp
from jax import lax
from jax.experimental import pallas as pl
from jax.experimental.pallas import tpu as pltpu
from jax.experimental.pallas import tpu_sc as plsc

N_SITES = 6400000
N_CRYSTALS = 100000
NC, NS = 2, 16
NW = NC * NS
SITES_PER_W = N_SITES // NW
PAD_SEG = 100352
SEG_PER_TILE = PAD_SEG // NS
FW = 2048
CHUNK = 4096
NCH = SITES_PER_W // CHUNK
REM = SITES_PER_W - NCH * CHUNK
UNROLL = 8

_MESH = plsc.VectorSubcoreMesh(core_axis_name="c", subcore_axis_name="s")


@functools.partial(
    pl.kernel,
    out_type=jax.ShapeDtypeStruct((NC * PAD_SEG,), jnp.float32),
    mesh=_MESH,
    scratch_types=[
        pltpu.VMEM_SHARED((PAD_SEG,), jnp.float32),
        pltpu.VMEM((PAD_SEG,), jnp.float32),
        pltpu.VMEM((CHUNK + 16,), jnp.int32),
        pltpu.VMEM((CHUNK,), jnp.float32),
        pltpu.VMEM((CHUNK + 16,), jnp.int32),
        pltpu.VMEM((CHUNK,), jnp.float32),
        pltpu.VMEM((FW,), jnp.int32),
        pltpu.VMEM((16,), jnp.int32),
        pltpu.VMEM((16,), jnp.int32),
        pltpu.SemaphoreType.DMA,
        pltpu.SemaphoreType.DMA,
    ],
    compiler_params=pltpu.CompilerParams(needs_layout_passes=False),
)
def _sc_partial(en_hbm, ids_hbm, zeros_hbm, out_hbm,
                acc, lacc, ids_b0, en_b0, ids_b1, en_b1,
                idxb, fb, lb, sem_i, sem_e):
    c = lax.axis_index("c")
    s = lax.axis_index("s")
    wid = c * NS + s
    base = wid * SITES_PER_W

    pltpu.sync_copy(zeros_hbm,
                    acc.at[pl.ds(s * SEG_PER_TILE, SEG_PER_TILE)])

    pltpu.sync_copy(ids_hbm.at[pl.ds(base, 16)], fb)
    pltpu.sync_copy(ids_hbm.at[pl.ds(base + SITES_PER_W - 16, 16)], lb)
    lo = fb[...][0]
    hi = lb[...][15]
    g0 = lo // FW
    nf = hi // FW - g0 + 1
    za0 = g0 * FW

    def zbody(j, _):
        lacc[pl.ds(za0 + j * 16, 16)] = jnp.zeros((16,), jnp.float32)
        return ()
    lax.fori_loop(0, nf * (FW // 16), zbody, ())

    iota = lax.iota(jnp.int32, 16)
    lane15 = iota == 15
    not15 = iota != 15

    def vreg(ids_b, en_b, j):
        idv = ids_b[pl.ds(j * 16, 16)]
        idv_up = ids_b[pl.ds(j * 16 + 1, 16)]
        env = en_b[pl.ds(j * 16, 16)]
        m_change = (idv != idv_up) & not15
        m_end = m_change | lane15
        cs = plsc.cumsum(env)
        plsc.addupdate_scatter(lacc, [idv], cs, mask=m_end)
        plsc.addupdate_scatter(lacc, [idv_up], -cs, mask=m_change)

    def process(ids_b, en_b, nvec):
        @functools.partial(plsc.parallel_loop, 0, nvec, unroll=UNROLL)
        def _(j):
            vreg(ids_b, en_b, j)

    rst = base + NCH * CHUNK
    pltpu.sync_copy(ids_hbm.at[pl.ds(rst, REM)], ids_b0.at[pl.ds(0, REM)])
    pltpu.sync_copy(en_hbm.at[pl.ds(rst, REM)], en_b0.at[pl.ds(0, REM)])
    process(ids_b0, en_b0, REM // 16)

    def start(gi, ids_b, en_b):
        st = base + gi * CHUNK
        pltpu.async_copy(ids_hbm.at[pl.ds(st, CHUNK)],
                         ids_b.at[pl.ds(0, CHUNK)], sem_i)
        pltpu.async_copy(en_hbm.at[pl.ds(st, CHUNK)], en_b, sem_e)

    def drain(ids_b, en_b):
        pltpu.make_async_copy(ids_hbm.at[pl.ds(0, CHUNK)],
                              ids_b.at[pl.ds(0, CHUNK)], sem_i).wait()
        pltpu.make_async_copy(en_hbm.at[pl.ds(0, CHUNK)], en_b, sem_e).wait()

    bufs = ((ids_b0, en_b0), (ids_b1, en_b1))
    start(0, *bufs[0])

    def ring(g2, _):
        for b in range(2):
            g = g2 * 2 + b
            drain(*bufs[b])

            @pl.when(g + 1 < NCH)
            def _():
                start(g + 1, *bufs[1 - b])

            process(*bufs[b], CHUNK // 16)
        return ()
    lax.fori_loop(0, NCH // 2, ring, ())

    plsc.subcore_barrier()

    def fbody(g, _):
        st = (g0 + g) * FW

        def ibody(j, _):
            idxb[pl.ds(j * 16, 16)] = (
                st + j * 16 + lax.iota(jnp.int32, 16))
            return ()
        lax.fori_loop(0, FW // 16, ibody, ())
        pltpu.sync_copy(lacc.at[pl.ds(st, FW)], acc.at[idxb], add=True)
        return ()
    lax.fori_loop(0, nf, fbody, ())

    plsc.subcore_barrier()

    off = s * SEG_PER_TILE
    pltpu.sync_copy(acc.at[pl.ds(off, SEG_PER_TILE)],
                    lacc.at[pl.ds(0, SEG_PER_TILE)])
    pltpu.sync_copy(lacc.at[pl.ds(0, SEG_PER_TILE)],
                    out_hbm.at[pl.ds(c * PAD_SEG + off, SEG_PER_TILE)])


SEG_PER_W = PAD_SEG // NW


@functools.partial(
    pl.kernel,
    out_type=jax.ShapeDtypeStruct((PAD_SEG,), jnp.float32),
    mesh=_MESH,
    scratch_types=[
        pltpu.VMEM((SEG_PER_W,), jnp.float32),
        pltpu.VMEM((SEG_PER_W,), jnp.float32),
    ],
)
def _sc_combine(partial_hbm, out_hbm, buf0, buf1):
    c = lax.axis_index("c")
    s = lax.axis_index("s")
    wid = c * NS + s
    off = wid * SEG_PER_W
    pltpu.sync_copy(partial_hbm.at[pl.ds(off, SEG_PER_W)], buf0)
    pltpu.sync_copy(partial_hbm.at[pl.ds(PAD_SEG + off, SEG_PER_W)], buf1)
    for j in range(SEG_PER_W // 16):
        sl = pl.ds(j * 16, 16)
        buf0[sl] = buf0[sl] + buf1[sl]
    pltpu.sync_copy(buf0, out_hbm.at[pl.ds(off, SEG_PER_W)])


def kernel(site_energy, segment_ids, num_crystals):
    en = site_energy.reshape(N_SITES)
    ids = segment_ids.reshape(N_SITES)
    zeros = jnp.zeros((SEG_PER_TILE,), jnp.float32)
    partial = _sc_partial(en, ids, zeros)
    pooled = _sc_combine(partial)
    return pooled[:N_CRYSTALS, None]
